# Initial kernel scaffold; baseline (speedup 1.0000x reference)
import jax
import jax.numpy as jnp
from jax.experimental import pallas as pl


def kernel(x10, x6, weight, gamma, beta):
    raise NotImplementedError("write your pallas kernel here")


# speedup vs baseline: 1.3160x; 1.3160x over previous
import functools

import jax
import jax.numpy as jnp
from jax import lax
from jax.experimental import pallas as pl
from jax.experimental.pallas import tpu as pltpu

_BN_EPS = 1e-3


def _fused_kernel(s_ref, w_ref, g_ref, b_ref, x_ref, o_ref,
                  xs_ref, gram_ref, s1_ref, wb_ref, bias_ref,
                  *, nt, inv_count):
    ph = pl.program_id(0)
    ni = pl.program_id(1)
    ti = pl.program_id(2)
    blk = ni * nt + ti

    @pl.when(ph == 0)
    def _stats():
        @pl.when(blk == 0)
        def _():
            gram_ref[...] = jnp.zeros_like(gram_ref)
            s1_ref[...] = jnp.zeros_like(s1_ref)

        xf = x_ref[...]
        xb = xf.astype(jnp.bfloat16)
        xs_ref[blk] = xb
        gram_ref[...] += lax.dot_general(
            xb, xb, (((1,), (1,)), ((), ())),
            preferred_element_type=jnp.float32)
        s1_ref[...] += jnp.sum(xf, axis=1, keepdims=True)

    @pl.when(ph == 1)
    def _apply():
        @pl.when(blk == 0)
        def _():
            hs = jnp.clip(s_ref[...] * (1.0 / 6.0) + 0.5, 0.0, 1.0)
            wp = w_ref[...] * hs
            mean = jnp.dot(wp, s1_ref[...],
                           preferred_element_type=jnp.float32) * inv_count
            gw = jnp.dot(wp, gram_ref[...],
                         preferred_element_type=jnp.float32)
            ey2 = jnp.sum(gw * wp, axis=1, keepdims=True) * inv_count
            var = ey2 - mean * mean
            inv_std = lax.rsqrt(var + _BN_EPS)
            scale = g_ref[...] * inv_std
            bias_ref[...] = b_ref[...] - mean * scale
            wb_ref[...] = (wp * scale).astype(jnp.bfloat16)

        y = jnp.dot(wb_ref[...], xs_ref[blk],
                    preferred_element_type=jnp.float32)
        o_ref[...] = y + bias_ref[...]


def kernel(x10, x6, weight, gamma, beta):
    n, c, h, w_sp = x6.shape
    c_out = weight.shape[0]
    hw = h * w_sp

    bhw = hw
    for cand in (32768, 16384, 8192):
        if hw > cand and hw % cand == 0:
            bhw = cand
            break
    nt = hw // bhw

    x3 = x6.reshape(n, c, hw)
    s = x10.reshape(1, c).astype(jnp.float32)
    w2d = weight.reshape(c_out, c).astype(jnp.float32)
    g = gamma.reshape(c_out, 1).astype(jnp.float32)
    b = beta.reshape(c_out, 1).astype(jnp.float32)

    def small(shape):
        return pl.BlockSpec(shape, lambda ph, ni, ti: (0, 0))

    out3 = pl.pallas_call(
        functools.partial(_fused_kernel, nt=nt, inv_count=1.0 / float(n * hw)),
        out_shape=jax.ShapeDtypeStruct((n, c_out, hw), jnp.float32),
        grid=(2, n, nt),
        in_specs=[
            small((1, c)),
            small((c_out, c)),
            small((c_out, 1)),
            small((c_out, 1)),
            pl.BlockSpec((pl.Squeezed(), c, bhw),
                         lambda ph, ni, ti: (ni * (1 - ph), 0, ti * (1 - ph))),
        ],
        out_specs=pl.BlockSpec((pl.Squeezed(), c_out, bhw),
                               lambda ph, ni, ti: (ni * ph, 0, ti * ph)),
        scratch_shapes=[
            pltpu.VMEM((n * nt, c, bhw), jnp.bfloat16),
            pltpu.VMEM((c, c), jnp.float32),
            pltpu.VMEM((c, 1), jnp.float32),
            pltpu.VMEM((c_out, c), jnp.bfloat16),
            pltpu.VMEM((c_out, 1), jnp.float32),
        ],
        compiler_params=pltpu.CompilerParams(
            dimension_semantics=("arbitrary", "arbitrary", "arbitrary"),
            vmem_limit_bytes=60 * 1024 * 1024),
    )(s, w2d, g, b, x3)

    return out3.reshape(n, c_out, h, w_sp)


# speedup vs baseline: 3.1960x; 2.4287x over previous
import functools

import jax
import jax.numpy as jnp
from jax import lax
from jax.experimental import pallas as pl
from jax.experimental.pallas import tpu as pltpu

_BN_EPS = 1e-3


def _fused_kernel(s_ref, w_ref, g_ref, b_ref, x_ref, o_ref,
                  xs_ref, gram_ref, s1_ref, wb_ref, bias_ref,
                  *, nt, inv_count):
    ph = pl.program_id(0)
    ni = pl.program_id(1)
    ti = pl.program_id(2)
    blk = ni * nt + ti

    @pl.when(ph == 0)
    def _stats():
        @pl.when(blk == 0)
        def _():
            gram_ref[...] = jnp.zeros_like(gram_ref)
            s1_ref[...] = jnp.zeros_like(s1_ref)

        xf3 = x_ref[...]
        c = xf3.shape[0]
        xb = xf3.reshape(c, -1).astype(jnp.bfloat16)
        xs_ref[blk] = xb
        gram_ref[...] += lax.dot_general(
            xb, xb, (((1,), (1,)), ((), ())),
            preferred_element_type=jnp.float32)
        s1_ref[...] += jnp.sum(xf3, axis=(1, 2))[:, None]

    @pl.when(ph == 1)
    def _apply():
        @pl.when(blk == 0)
        def _():
            hs = jnp.clip(s_ref[...] * (1.0 / 6.0) + 0.5, 0.0, 1.0)
            wp = w_ref[...] * hs
            mean = jnp.dot(wp, s1_ref[...],
                           preferred_element_type=jnp.float32) * inv_count
            gw = jnp.dot(wp, gram_ref[...],
                         preferred_element_type=jnp.float32)
            ey2 = jnp.sum(gw * wp, axis=1, keepdims=True) * inv_count
            var = ey2 - mean * mean
            inv_std = lax.rsqrt(var + _BN_EPS)
            scale = g_ref[...] * inv_std
            bias_ref[...] = (b_ref[...] - mean * scale)[:, :, None]
            wb_ref[...] = (wp * scale).astype(jnp.bfloat16)

        y = jnp.dot(wb_ref[...], xs_ref[blk],
                    preferred_element_type=jnp.float32)
        co, bh, w = o_ref.shape
        o_ref[...] = y.reshape(co, bh, w) + bias_ref[...]


def kernel(x10, x6, weight, gamma, beta):
    n, c, h, w_sp = x6.shape
    c_out = weight.shape[0]

    bh = h
    for cand in (128, 64, 32, 16, 8):
        if h > cand and h % cand == 0:
            bh = cand
            break
    nt = h // bh

    s = x10.reshape(1, c).astype(jnp.float32)
    w2d = weight.reshape(c_out, c).astype(jnp.float32)
    g = gamma.reshape(c_out, 1).astype(jnp.float32)
    b = beta.reshape(c_out, 1).astype(jnp.float32)

    def small(shape):
        return pl.BlockSpec(shape, lambda ph, ni, ti: (0, 0))

    out = pl.pallas_call(
        functools.partial(_fused_kernel, nt=nt,
                          inv_count=1.0 / float(n * h * w_sp)),
        out_shape=jax.ShapeDtypeStruct((n, c_out, h, w_sp), jnp.float32),
        grid=(2, n, nt),
        in_specs=[
            small((1, c)),
            small((c_out, c)),
            small((c_out, 1)),
            small((c_out, 1)),
            pl.BlockSpec((pl.Squeezed(), c, bh, w_sp),
                         lambda ph, ni, ti: (ni * (1 - ph), 0, ti * (1 - ph), 0)),
        ],
        out_specs=pl.BlockSpec((pl.Squeezed(), c_out, bh, w_sp),
                               lambda ph, ni, ti: (ni * ph, 0, ti * ph, 0)),
        scratch_shapes=[
            pltpu.VMEM((n * nt, c, bh * w_sp), jnp.bfloat16),
            pltpu.VMEM((c, c), jnp.float32),
            pltpu.VMEM((c, 1), jnp.float32),
            pltpu.VMEM((c_out, c), jnp.bfloat16),
            pltpu.VMEM((c_out, 1, 1), jnp.float32),
        ],
        compiler_params=pltpu.CompilerParams(
            dimension_semantics=("arbitrary", "arbitrary", "arbitrary"),
            vmem_limit_bytes=60 * 1024 * 1024),
    )(s, w2d, g, b, x6)

    return out


# speedup vs baseline: 3.9038x; 1.2215x over previous
import functools

import jax
import jax.numpy as jnp
from jax import lax
from jax.experimental import pallas as pl
from jax.experimental.pallas import tpu as pltpu

_BN_EPS = 1e-3


def _fused_kernel(s_ref, w_ref, g_ref, b_ref, x_ref, o_ref,
                  xs_ref, gram_ref, s1_ref, wb_ref, bias_ref,
                  *, nt, inv_count):
    ph = pl.program_id(0)
    ni = pl.program_id(1)
    ti = pl.program_id(2)
    blk = ni * nt + ti

    @pl.when(ph == 0)
    def _stats():
        @pl.when(blk == 0)
        def _():
            gram_ref[...] = jnp.zeros_like(gram_ref)
            s1_ref[...] = jnp.zeros_like(s1_ref)

        xf3 = x_ref[...]
        c = xf3.shape[0]
        xb = xf3.astype(jnp.bfloat16).reshape(c, -1)
        xs_ref[blk] = xb
        gram_ref[...] += lax.dot_general(
            xb, xb, (((1,), (1,)), ((), ())),
            preferred_element_type=jnp.float32)
        s1_ref[...] += jnp.sum(xb, axis=1, keepdims=True,
                               dtype=jnp.float32)

    @pl.when(ph == 1)
    def _apply():
        @pl.when(blk == 0)
        def _():
            hs = jnp.clip(s_ref[...] * (1.0 / 6.0) + 0.5, 0.0, 1.0)
            wp = w_ref[...] * hs
            mean = jnp.dot(wp, s1_ref[...],
                           preferred_element_type=jnp.float32) * inv_count
            gw = jnp.dot(wp, gram_ref[...],
                         preferred_element_type=jnp.float32)
            ey2 = jnp.sum(gw * wp, axis=1, keepdims=True) * inv_count
            var = ey2 - mean * mean
            inv_std = lax.rsqrt(var + _BN_EPS)
            scale = g_ref[...] * inv_std
            bias_ref[...] = (b_ref[...] - mean * scale)[:, :, None]
            wb_ref[...] = (wp * scale).astype(jnp.bfloat16)

        y = jnp.dot(wb_ref[...], xs_ref[blk],
                    preferred_element_type=jnp.float32)
        co, bh, w = o_ref.shape
        o_ref[...] = y.reshape(co, bh, w) + bias_ref[...]


def kernel(x10, x6, weight, gamma, beta):
    n, c, h, w_sp = x6.shape
    c_out = weight.shape[0]

    bh = h
    nt = h // bh

    s = x10.reshape(1, c).astype(jnp.float32)
    w2d = weight.reshape(c_out, c).astype(jnp.float32)
    g = gamma.reshape(c_out, 1).astype(jnp.float32)
    b = beta.reshape(c_out, 1).astype(jnp.float32)

    def small(shape):
        return pl.BlockSpec(shape, lambda ph, ni, ti: (0, 0))

    out = pl.pallas_call(
        functools.partial(_fused_kernel, nt=nt,
                          inv_count=1.0 / float(n * h * w_sp)),
        out_shape=jax.ShapeDtypeStruct((n, c_out, h, w_sp), jnp.float32),
        grid=(2, n, nt),
        in_specs=[
            small((1, c)),
            small((c_out, c)),
            small((c_out, 1)),
            small((c_out, 1)),
            pl.BlockSpec((pl.Squeezed(), c, bh, w_sp),
                         lambda ph, ni, ti: (ni * (1 - ph), 0, ti * (1 - ph), 0)),
        ],
        out_specs=pl.BlockSpec((pl.Squeezed(), c_out, bh, w_sp),
                               lambda ph, ni, ti: (ni * ph, 0, ti * ph, 0)),
        scratch_shapes=[
            pltpu.VMEM((n * nt, c, bh * w_sp), jnp.bfloat16),
            pltpu.VMEM((c, c), jnp.float32),
            pltpu.VMEM((c, 1), jnp.float32),
            pltpu.VMEM((c_out, c), jnp.bfloat16),
            pltpu.VMEM((c_out, 1, 1), jnp.float32),
        ],
        compiler_params=pltpu.CompilerParams(
            dimension_semantics=("arbitrary", "arbitrary", "arbitrary"),
            vmem_limit_bytes=60 * 1024 * 1024),
    )(s, w2d, g, b, x6)

    return out
